# baseline (speedup 1.0000x reference)
import functools
import math

import jax
import jax.numpy as jnp
from jax import lax
from jax.experimental import pallas as pl
from jax.experimental.pallas import tpu as pltpu
from jax.experimental.pallas import tpu_sc as plsc

S = 64.0
MARGIN = 0.5
COS_M = math.cos(MARGIN)
SIN_M = math.sin(MARGIN)
THETA = math.cos(math.pi - MARGIN)
SINMM = math.sin(math.pi - MARGIN) * MARGIN
C1 = S / math.log(2.0)

LANES = 16


def _make_sc_gather(B, V, num_cores, num_subcores):
    nw = num_cores * num_subcores
    b_per_w = B // nw
    assert b_per_w % LANES == 0 and B % (8 * nw) == 0

    mesh = plsc.VectorSubcoreMesh(core_axis_name="c", subcore_axis_name="s")

    @functools.partial(
        pl.kernel,
        out_type=jax.ShapeDtypeStruct((B,), jnp.float32),
        mesh=mesh,
        scratch_types=[
            pltpu.VMEM((b_per_w,), jnp.int32),
            pltpu.VMEM((b_per_w,), jnp.int32),
            pltpu.VMEM((b_per_w,), jnp.float32),
            pltpu.SemaphoreType.DMA,
        ],
    )
    def sc_gather(labels_hbm, logits_flat_hbm, t_hbm, lbl_v, idx_v, val_v, sem):
        wid = lax.axis_index("s") * num_cores + lax.axis_index("c")
        base = wid * b_per_w
        pltpu.sync_copy(labels_hbm.at[pl.ds(base, b_per_w)], lbl_v)
        for s in range(b_per_w // LANES):
            lbl = lbl_v[pl.ds(s * LANES, LANES)]
            lbl = jnp.where(lbl < 0, 0, lbl)
            rows = (base + s * LANES) + lax.iota(jnp.int32, LANES)
            idx_v[pl.ds(s * LANES, LANES)] = rows * jnp.int32(V) + lbl
        pltpu.async_copy(logits_flat_hbm.at[idx_v], val_v, sem).wait()
        pltpu.sync_copy(val_v, t_hbm.at[pl.ds(base, b_per_w)])

    return sc_gather


def _tc_body(t_ref, labels_ref, *rest, nsteps, B, V, BR, nsplit):
    logit_refs = rest[:nsplit]
    out_ref = rest[nsplit]
    rowsum_ref = rest[nsplit + 1]
    i = pl.program_id(0)
    rows_per_op = B // nsplit
    for k in range(nsplit):
        x = logit_refs[k][...]
        cols = lax.broadcasted_iota(jnp.int32, (BR, V), 1)
        x = jnp.where(cols < V, x, -1.0)
        e = jnp.exp2(C1 * x - C1)
        rowsum_ref[pl.ds(k * rows_per_op + i * BR, BR), :] = jnp.sum(
            e, axis=1, keepdims=True
        )

    @pl.when(i == nsteps - 1)
    def _epilogue():
        rowsum = rowsum_ref[...]
        t = t_ref[...]
        labels = labels_ref[...]
        sin_t = jnp.sqrt(jnp.maximum(1.0 - t * t, 0.0))
        new_t = jnp.where(t > THETA, t * COS_M - sin_t * SIN_M, t - SINMM)
        new_t = jnp.where(labels != -1, new_t, t)
        adj = rowsum - jnp.exp2(C1 * t - C1) + jnp.exp2(C1 * new_t - C1)
        adj = jnp.maximum(adj, 1e-35)
        lse = S + jnp.log(adj)
        out_ref[0, 0] = jnp.sum(lse - S * new_t) * (1.0 / B)


def _tc_loss(logits, t, labels_i32, BR=8, nsplit=4):
    B, V = logits.shape
    rows_per_op = B // nsplit
    nsteps = rows_per_op // BR
    body = functools.partial(
        _tc_body, nsteps=nsteps, B=B, V=V, BR=BR, nsplit=nsplit
    )
    blk_per_op = rows_per_op // BR

    def make_map(k):
        return lambda i: (k * blk_per_op + i, 0)

    out = pl.pallas_call(
        body,
        grid=(nsteps,),
        in_specs=[
            pl.BlockSpec((B, 1), lambda i: (0, 0)),
            pl.BlockSpec((B, 1), lambda i: (0, 0)),
        ]
        + [pl.BlockSpec((BR, V), make_map(k)) for k in range(nsplit)],
        out_specs=pl.BlockSpec(memory_space=pltpu.SMEM),
        out_shape=jax.ShapeDtypeStruct((1, 1), jnp.float32),
        scratch_shapes=[pltpu.VMEM((B, 1), jnp.float32)],
    )(t.reshape(B, 1), labels_i32.reshape(B, 1), *([logits] * nsplit))
    return out[0, 0]


def kernel(logits, labels):
    B, V = logits.shape
    labels_i32 = labels.astype(jnp.int32)
    info = plsc.get_sparse_core_info()
    sc_gather = _make_sc_gather(B, V, info.num_cores, info.num_subcores)
    t = sc_gather(labels_i32, logits.reshape(B * V))
    return _tc_loss(logits, t, labels_i32)


# speedup vs baseline: 2.0280x; 2.0280x over previous
import functools
import math

import jax
import jax.numpy as jnp
from jax import lax
from jax.experimental import pallas as pl
from jax.experimental.pallas import tpu as pltpu

S = 64.0
MARGIN = 0.5
COS_M = math.cos(MARGIN)
SIN_M = math.sin(MARGIN)
THETA = math.cos(math.pi - MARGIN)
SINMM = math.sin(math.pi - MARGIN) * MARGIN
C1 = S / math.log(2.0)


def _tc_body(labels_blk_ref, labels_ref, logits_ref, out_ref, rowsum_ref, t_ref,
             *, nsteps, B, V, BR):
    i = pl.program_id(0)
    x = logits_ref[...]
    cols = lax.broadcasted_iota(jnp.int32, (BR, V), 1)
    lbl = labels_blk_ref[...]
    safe_lbl = jnp.where(lbl < 0, 0, lbl)
    x = jnp.where(cols < V, x, -1.0)
    e = jnp.exp2(C1 * x - C1)
    rowsum_ref[pl.ds(i * BR, BR), :] = jnp.sum(e, axis=1, keepdims=True)
    t_val = jnp.sum(jnp.where(cols == safe_lbl, x, 0.0), axis=1, keepdims=True)
    t_ref[pl.ds(i * BR, BR), :] = t_val

    @pl.when(i == nsteps - 1)
    def _epilogue():
        rowsum = rowsum_ref[...]
        t = t_ref[...]
        labels = labels_ref[...]
        sin_t = jnp.sqrt(jnp.maximum(1.0 - t * t, 0.0))
        new_t = jnp.where(t > THETA, t * COS_M - sin_t * SIN_M, t - SINMM)
        new_t = jnp.where(labels != -1, new_t, t)
        adj = rowsum - jnp.exp2(C1 * t - C1) + jnp.exp2(C1 * new_t - C1)
        adj = jnp.maximum(adj, 1e-35)
        lse = S + jnp.log(adj)
        out_ref[0, 0] = jnp.sum(lse - S * new_t) * (1.0 / B)


def kernel(logits, labels):
    B, V = logits.shape
    labels_i32 = labels.astype(jnp.int32).reshape(B, 1)
    BR = 16
    nsteps = B // BR
    body = functools.partial(_tc_body, nsteps=nsteps, B=B, V=V, BR=BR)
    out = pl.pallas_call(
        body,
        grid=(nsteps,),
        in_specs=[
            pl.BlockSpec((BR, 1), lambda i: (i, 0)),
            pl.BlockSpec((B, 1), lambda i: (0, 0)),
            pl.BlockSpec((BR, V), lambda i: (i, 0)),
        ],
        out_specs=pl.BlockSpec(memory_space=pltpu.SMEM),
        out_shape=jax.ShapeDtypeStruct((1, 1), jnp.float32),
        scratch_shapes=[
            pltpu.VMEM((B, 1), jnp.float32),
            pltpu.VMEM((B, 1), jnp.float32),
        ],
    )(labels_i32, labels_i32, logits)
    return out[0, 0]


# speedup vs baseline: 2.1350x; 1.0527x over previous
import functools
import math

import jax
import jax.numpy as jnp
from jax import lax
from jax.experimental import pallas as pl
from jax.experimental.pallas import tpu as pltpu

S = 64.0
MARGIN = 0.5
COS_M = math.cos(MARGIN)
SIN_M = math.sin(MARGIN)
THETA = math.cos(math.pi - MARGIN)
SINMM = math.sin(math.pi - MARGIN) * MARGIN
C1 = S / math.log(2.0)


def _tc_body(labels_blk_ref, labels_ref, logits_ref, out_ref, rowsum_ref, t_ref,
             *, nsteps, B, V, BR):
    i = pl.program_id(0)
    x = logits_ref[...]
    cols = lax.broadcasted_iota(jnp.int32, (BR, V), 1)
    lbl = labels_blk_ref[...]
    safe_lbl = jnp.where(lbl < 0, 0, lbl)
    x = jnp.where(cols < V, x, -1.0)
    e = jnp.exp2(C1 * x - C1)
    rowsum_ref[pl.ds(i * BR, BR), :] = jnp.sum(e, axis=1, keepdims=True)
    t_val = jnp.sum(jnp.where(cols == safe_lbl, x, 0.0), axis=1, keepdims=True)
    t_ref[pl.ds(i * BR, BR), :] = t_val

    @pl.when(i == nsteps - 1)
    def _epilogue():
        rowsum = rowsum_ref[...]
        t = t_ref[...]
        labels = labels_ref[...]
        sin_t = jnp.sqrt(jnp.maximum(1.0 - t * t, 0.0))
        new_t = jnp.where(t > THETA, t * COS_M - sin_t * SIN_M, t - SINMM)
        new_t = jnp.where(labels != -1, new_t, t)
        adj = rowsum - jnp.exp2(C1 * t - C1) + jnp.exp2(C1 * new_t - C1)
        adj = jnp.maximum(adj, 1e-35)
        lse = S + jnp.log(adj)
        out_ref[0, 0] = jnp.sum(lse - S * new_t) * (1.0 / B)


def kernel(logits, labels):
    B, V = logits.shape
    labels_i32 = labels.astype(jnp.int32).reshape(B, 1)
    BR = 32
    nsteps = B // BR
    body = functools.partial(_tc_body, nsteps=nsteps, B=B, V=V, BR=BR)
    out = pl.pallas_call(
        body,
        grid=(nsteps,),
        in_specs=[
            pl.BlockSpec((BR, 1), lambda i: (i, 0)),
            pl.BlockSpec((B, 1), lambda i: (0, 0)),
            pl.BlockSpec((BR, V), lambda i: (i, 0)),
        ],
        out_specs=pl.BlockSpec(memory_space=pltpu.SMEM),
        out_shape=jax.ShapeDtypeStruct((1, 1), jnp.float32),
        scratch_shapes=[
            pltpu.VMEM((B, 1), jnp.float32),
            pltpu.VMEM((B, 1), jnp.float32),
        ],
    )(labels_i32, labels_i32, logits)
    return out[0, 0]


# speedup vs baseline: 2.1453x; 1.0049x over previous
import functools
import math

import jax
import jax.numpy as jnp
from jax import lax
from jax.experimental import pallas as pl
from jax.experimental.pallas import tpu as pltpu

S = 64.0
MARGIN = 0.5
COS_M = math.cos(MARGIN)
SIN_M = math.sin(MARGIN)
THETA = math.cos(math.pi - MARGIN)
SINMM = math.sin(math.pi - MARGIN) * MARGIN
C1 = S / math.log(2.0)


def _tc_body(labels_ref, *rest, nsteps, B, V, BR, nsplit):
    logit_refs = rest[:nsplit]
    out_ref = rest[nsplit]
    rowsum_ref = rest[nsplit + 1]
    t_ref = rest[nsplit + 2]
    i = pl.program_id(0)
    rows_per_op = B // nsplit
    for k in range(nsplit):
        x = logit_refs[k][...]
        row0 = k * rows_per_op + i * BR
        cols = lax.broadcasted_iota(jnp.int32, (BR, V), 1)
        lbl = labels_ref[pl.ds(row0, BR), :]
        safe_lbl = jnp.where(lbl < 0, 0, lbl)
        x = jnp.where(cols < V, x, -1.0)
        e = jnp.exp2(C1 * x - C1)
        rowsum_ref[pl.ds(row0, BR), :] = jnp.sum(e, axis=1, keepdims=True)
        t_val = jnp.sum(jnp.where(cols == safe_lbl, x, 0.0), axis=1, keepdims=True)
        t_ref[pl.ds(row0, BR), :] = t_val

    @pl.when(i == nsteps - 1)
    def _epilogue():
        rowsum = rowsum_ref[...]
        t = t_ref[...]
        labels = labels_ref[...]
        sin_t = jnp.sqrt(jnp.maximum(1.0 - t * t, 0.0))
        new_t = jnp.where(t > THETA, t * COS_M - sin_t * SIN_M, t - SINMM)
        new_t = jnp.where(labels != -1, new_t, t)
        adj = rowsum - jnp.exp2(C1 * t - C1) + jnp.exp2(C1 * new_t - C1)
        adj = jnp.maximum(adj, 1e-35)
        lse = S + jnp.log(adj)
        out_ref[0, 0] = jnp.sum(lse - S * new_t) * (1.0 / B)


def kernel(logits, labels):
    B, V = logits.shape
    labels_i32 = labels.astype(jnp.int32).reshape(B, 1)
    BR = 8
    nsplit = 4
    rows_per_op = B // nsplit
    nsteps = rows_per_op // BR
    blk_per_op = rows_per_op // BR
    body = functools.partial(
        _tc_body, nsteps=nsteps, B=B, V=V, BR=BR, nsplit=nsplit
    )

    def make_map(k):
        return lambda i: (k * blk_per_op + i, 0)

    out = pl.pallas_call(
        body,
        grid=(nsteps,),
        in_specs=[pl.BlockSpec((B, 1), lambda i: (0, 0))]
        + [pl.BlockSpec((BR, V), make_map(k)) for k in range(nsplit)],
        out_specs=pl.BlockSpec(memory_space=pltpu.SMEM),
        out_shape=jax.ShapeDtypeStruct((1, 1), jnp.float32),
        scratch_shapes=[
            pltpu.VMEM((B, 1), jnp.float32),
            pltpu.VMEM((B, 1), jnp.float32),
        ],
    )(labels_i32, *([logits] * nsplit))
    return out[0, 0]
